# Initial kernel scaffold; baseline (speedup 1.0000x reference)
import jax
import jax.numpy as jnp
from jax.experimental import pallas as pl


def kernel(x, W_wdir, W_weather, W_day, W_hour):
    raise NotImplementedError("write your pallas kernel here")


# speedup vs baseline: 6.5686x; 6.5686x over previous
import functools

import jax
import jax.numpy as jnp
from jax import lax
from jax.experimental import pallas as pl
from jax.experimental.pallas import tpu as pltpu
from jax.experimental.pallas import tpu_sc as plsc

_NC = 2
_NS = 16
_NW = _NC * _NS
_L = 16

_B_BLK = 2048


def _make_sc_call(n_pos, widths, interpret=False):
    w0, w1, w2, w3 = widths
    d_out = w0 + w1 + w2 + w3
    assert n_pos % (_NW * _B_BLK) == 0
    p_per_w = n_pos // _NW
    n_blk = p_per_w // _B_BLK
    grp_per_blk = _B_BLK // _L

    def body(x_hbm, t0_hbm, t1_hbm, t2_hbm, t3_hbm, out_hbm,
             t0v, t1v, t2v, t3v, xv, ov):
        wid = lax.axis_index("s") * _NC + lax.axis_index("c")
        base = wid * p_per_w
        pltpu.sync_copy(t0_hbm, t0v)
        pltpu.sync_copy(t1_hbm, t1v)
        pltpu.sync_copy(t2_hbm, t2v)
        pltpu.sync_copy(t3_hbm, t3v)

        def blk(b, carry):
            start = base + b * _B_BLK
            pltpu.sync_copy(x_hbm.at[pl.ds(start * 4, _B_BLK * 4)], xv)

            def grp(g, c2):
                iota = lax.iota(jnp.int32, _L)
                xb = g * (4 * _L) + iota * 4
                x0 = plsc.load_gather(xv, [xb])
                x1 = plsc.load_gather(xv, [xb + 1])
                x2 = plsc.load_gather(xv, [xb + 2])
                x3 = plsc.load_gather(xv, [xb + 3])
                ob = g * (d_out * _L) + iota * d_out
                a0 = x0 * w0
                a1 = x1 * w1
                a2 = x2 * w2
                a3 = x3 * w3
                for j in range(w0):
                    plsc.store_scatter(ov, [ob + j],
                                       plsc.load_gather(t0v, [a0 + j]))
                for j in range(w1):
                    plsc.store_scatter(ov, [ob + (w0 + j)],
                                       plsc.load_gather(t1v, [a1 + j]))
                for j in range(w2):
                    plsc.store_scatter(ov, [ob + (w0 + w1 + j)],
                                       plsc.load_gather(t2v, [a2 + j]))
                for j in range(w3):
                    plsc.store_scatter(ov, [ob + (w0 + w1 + w2 + j)],
                                       plsc.load_gather(t3v, [a3 + j]))
                return c2

            lax.fori_loop(0, grp_per_blk, grp, 0)
            pltpu.sync_copy(ov, out_hbm.at[pl.ds(start * d_out,
                                                 _B_BLK * d_out)])
            return carry

        lax.fori_loop(0, n_blk, blk, 0)

    mesh = plsc.VectorSubcoreMesh(core_axis_name="c", subcore_axis_name="s",
                                  num_cores=_NC, num_subcores=_NS)
    return pl.kernel(
        body,
        out_type=jax.ShapeDtypeStruct((n_pos * d_out,), jnp.float32),
        mesh=mesh,
        scratch_types=[
            pltpu.VMEM((11 * w0,), jnp.float32),
            pltpu.VMEM((18 * w1,), jnp.float32),
            pltpu.VMEM((24 * w2,), jnp.float32),
            pltpu.VMEM((7 * w3,), jnp.float32),
            pltpu.VMEM((_B_BLK * 4,), jnp.int32),
            pltpu.VMEM((_B_BLK * 15,), jnp.float32),
        ],
        compiler_params=pltpu.CompilerParams(needs_layout_passes=False),
        interpret=interpret,
    )


def kernel(x, W_wdir, W_weather, W_day, W_hour):
    b, t, _ = x.shape
    n_pos = b * t
    widths = (W_wdir.shape[1], W_weather.shape[1],
              W_day.shape[1], W_hour.shape[1])
    d_out = sum(widths)
    call = _make_sc_call(n_pos, widths)
    out = call(x.reshape(-1).astype(jnp.int32),
               W_wdir.reshape(-1), W_weather.reshape(-1),
               W_day.reshape(-1), W_hour.reshape(-1))
    return out.reshape(b, t, d_out)
